# Initial kernel scaffold; baseline (speedup 1.0000x reference)
import jax
import jax.numpy as jnp
from jax.experimental import pallas as pl


def kernel(xyz, points, W0, b0, gamma0, beta0, W1, b1, gamma1, beta1, W2, b2, gamma2, beta2):
    raise NotImplementedError("write your pallas kernel here")


# speedup vs baseline: 12.3300x; 12.3300x over previous
import functools

import jax
import jax.numpy as jnp
from jax import lax
from jax.experimental import pallas as pl
from jax.experimental.pallas import tpu as pltpu
from jax.experimental.pallas import tpu_sc as plsc

_B, _N, _D = 8, 2048, 128
_NP, _NS = 512, 32
_R2 = 0.2 ** 2
_CIN = 3 + _D
_CPAD = 144
_NTILES = 32
_ROWS_PER = (_B * _NP) // _NTILES
_CNT = _B * _NP * _NS
_TR = 2048
_NT = (_NP * _NS) // _TR


def _fps_body(xyzp_ref, out_ref):
    x = xyzp_ref[0]
    y = xyzp_ref[1]
    z = xyzp_ref[2]
    lane = lax.broadcasted_iota(jnp.int32, (_B, _N), 1)
    zero = jnp.float32(0)

    def body(i, carry):
        distance, farthest = carry
        oh = lane == farthest
        cx = jnp.sum(jnp.where(oh, x, zero), axis=1, keepdims=True)
        cy = jnp.sum(jnp.where(oh, y, zero), axis=1, keepdims=True)
        cz = jnp.sum(jnp.where(oh, z, zero), axis=1, keepdims=True)
        out_ref[pl.ds(i, 1)] = jnp.concatenate([cx, cy, cz], axis=1)[None]
        dx = x - cx
        dy = y - cy
        dz = z - cz
        dist = (dx * dx + dy * dy) + dz * dz
        distance = jnp.minimum(distance, dist)
        m = jnp.max(distance, axis=1, keepdims=True)
        farthest = jnp.min(jnp.where(distance == m, lane, _N), axis=1,
                           keepdims=True)
        return distance, farthest

    lax.fori_loop(0, _NP, body,
                  (jnp.full((_B, _N), 1e10, jnp.float32),
                   jnp.zeros((_B, 1), jnp.int32)))


def _run_fps(xyzp):
    return pl.pallas_call(
        _fps_body,
        out_shape=jax.ShapeDtypeStruct((_NP, _B, 3), jnp.float32),
    )(xyzp)


def _dist_body(nx_ref, xt_ref, out_ref):
    nx = nx_ref[0]
    xt = xt_ref[0]
    mm = jnp.dot(nx, xt, preferred_element_type=jnp.float32)
    s1 = jnp.sum(nx * nx, axis=1, keepdims=True)
    s2 = jnp.sum(xt * xt, axis=0, keepdims=True)
    d = -2.0 * mm
    d = d + s1
    d = d + s2
    out_ref[0] = d


def _run_dist(new_xyz, xyz_t):
    return pl.pallas_call(
        _dist_body,
        grid=(_B,),
        in_specs=[pl.BlockSpec((1, _NP, 3), lambda b: (b, 0, 0)),
                  pl.BlockSpec((1, 3, _N), lambda b: (b, 0, 0))],
        out_specs=pl.BlockSpec((1, _NP, _N), lambda b: (b, 0, 0)),
        out_shape=jax.ShapeDtypeStruct((_B, _NP, _N), jnp.float32),
    )(new_xyz, xyz_t)


_GRP = 4
_U = 4


def _ball_body(dist_h, xh, yh, zh, pts_h, outp_h, outx_h,
               xb, yb, zb, db0, db1, slots, gidx0, gidx1, gxyz0, gxyz1,
               rows0, rows1, sem0, sem1, gsem0, gsem1, osem0, osem1):
    wid = lax.axis_index("s") * 2 + lax.axis_index("c")
    b = wid // (_NP // _ROWS_PER)
    pltpu.sync_copy(xh.at[b], xb)
    pltpu.sync_copy(yh.at[b], yb)
    pltpu.sync_copy(zh.at[b], zb)

    iota16 = lax.iota(jnp.int32, 16)
    r2 = jnp.float32(_R2)
    nvec = _N // 16
    base_row = wid * _ROWS_PER
    last_row = _B * _NP - 1
    off = b * _N

    pltpu.async_copy(dist_h.at[base_row], db0, sem0)

    def scan_row(dbuf):
        def cond(carry):
            j, cnt, _fv = carry
            return jnp.logical_and(j < nvec, cnt < _NS)

        def step(carry):
            j, cnt, fv = carry
            cntv = jnp.full((16,), 0, jnp.int32) + cnt
            for u in range(_U):
                base = (j + u) * 16
                d = dbuf[pl.ds(base, 16)]
                m = d <= r2
                mi = m.astype(jnp.int32)
                pos = plsc.cumsum(mi) + cntv
                wmask = jnp.logical_and(m, pos <= _NS)
                slot = jnp.clip(pos - 1, 0, _NS - 1)
                plsc.store_scatter(slots, [slot], base + iota16, mask=wmask)
                fv = jnp.minimum(fv, jnp.where(m, base + iota16, _N - 1))
                cntv = cntv + plsc.all_reduce_population_count(m)
            return j + _U, jnp.max(cntv), fv

        _, cnt, fv = lax.while_loop(
            cond, step,
            (jnp.int32(0), jnp.int32(0),
             jnp.full((16,), _N - 1, jnp.int32)))
        return cnt, jnp.min(fv)

    def handle_row(i, q, dbuf, mysem, obuf, osem, gidx, gxyz):
        row = base_row + i
        pltpu.make_async_copy(dist_h.at[row], dbuf, mysem).wait()
        nxt = jnp.minimum(row + 1, last_row)
        pltpu.async_copy(dist_h.at[nxt], obuf, osem)
        cnt, first = scan_row(dbuf)
        firstv = jnp.full((16,), 0, jnp.int32) + first
        for h in range(_NS // 16):
            lanes = iota16 + h * 16
            cur = slots[pl.ds(h * 16, 16)]
            cur = jnp.where(lanes < cnt, cur, firstv)
            gidx[pl.ds(q * _NS + h * 16, 16)] = cur + off
            gx = plsc.load_gather(xb, [cur])
            gy = plsc.load_gather(yb, [cur])
            gz = plsc.load_gather(zb, [cur])
            gpos = (q * _NS + lanes) * 3
            plsc.store_scatter(gxyz, [gpos], gx)
            plsc.store_scatter(gxyz, [gpos + 1], gy)
            plsc.store_scatter(gxyz, [gpos + 2], gz)

    ngrp = _ROWS_PER // _GRP
    par = ((gidx0, gxyz0, rows0, gsem0, osem0),
           (gidx1, gxyz1, rows1, gsem1, osem1))

    def scan_group(g, parity):
        gidx, gxyz, _rows, _gsem, _osem = par[parity]
        for q in range(_GRP):
            if q % 2 == 0:
                handle_row(_GRP * g + q, q, db0, sem0, db1, sem1, gidx, gxyz)
            else:
                handle_row(_GRP * g + q, q, db1, sem1, db0, sem0, gidx, gxyz)

    def fire_gather(parity):
        gidx, _gxyz, rows, gsem, _osem = par[parity]
        pltpu.async_copy(pts_h.at[gidx], rows, gsem)

    def fire_out(g, parity):
        gidx, gxyz, rows, gsem, osem = par[parity]
        gout = wid * ngrp + g
        pltpu.make_async_copy(pts_h.at[gidx], rows, gsem).wait()
        pltpu.async_copy(rows, outp_h.at[gout], osem)
        pltpu.async_copy(gxyz, outx_h.at[gout], osem)

    def drain_out(parity):
        _gidx, gxyz, rows, _gsem, osem = par[parity]
        pltpu.make_async_copy(rows, outp_h.at[0], osem).wait()
        pltpu.make_async_copy(gxyz, outx_h.at[0], osem).wait()

    gout0 = wid * ngrp
    pltpu.async_copy(rows0, outp_h.at[gout0], osem0)
    pltpu.async_copy(gxyz0, outx_h.at[gout0], osem0)
    pltpu.async_copy(rows1, outp_h.at[gout0 + 1], osem1)
    pltpu.async_copy(gxyz1, outx_h.at[gout0 + 1], osem1)

    def pair_body(p, _):
        g0 = 2 * p
        drain_out(0)
        scan_group(g0, 0)
        fire_gather(0)
        drain_out(1)
        scan_group(g0 + 1, 1)
        fire_gather(1)
        fire_out(g0, 0)
        fire_out(g0 + 1, 1)
        return 0

    lax.fori_loop(0, ngrp // 2, pair_body, 0)
    drain_out(0)
    drain_out(1)
    pltpu.make_async_copy(dist_h.at[0], db0, sem0).wait()


@functools.lru_cache(maxsize=1)
def _ball_kernel():
    return functools.partial(
        pl.kernel,
        out_type=[
            jax.ShapeDtypeStruct((_B * _NP // _GRP, _GRP * _NS, _D),
                                 jnp.float32),
            jax.ShapeDtypeStruct((_B * _NP // _GRP, _GRP * _NS * 3),
                                 jnp.float32),
        ],
        mesh=plsc.VectorSubcoreMesh(core_axis_name="c", subcore_axis_name="s"),
        compiler_params=pltpu.CompilerParams(needs_layout_passes=False),
        scratch_types=[
            pltpu.VMEM((_N,), jnp.float32),
            pltpu.VMEM((_N,), jnp.float32),
            pltpu.VMEM((_N,), jnp.float32),
            pltpu.VMEM((_N,), jnp.float32),
            pltpu.VMEM((_N,), jnp.float32),
            pltpu.VMEM((_NS,), jnp.int32),
            pltpu.VMEM((_GRP * _NS,), jnp.int32),
            pltpu.VMEM((_GRP * _NS,), jnp.int32),
            pltpu.VMEM((_GRP * _NS * 3,), jnp.float32),
            pltpu.VMEM((_GRP * _NS * 3,), jnp.float32),
            pltpu.VMEM((_GRP * _NS, _D), jnp.float32),
            pltpu.VMEM((_GRP * _NS, _D), jnp.float32),
            pltpu.SemaphoreType.DMA,
            pltpu.SemaphoreType.DMA,
            pltpu.SemaphoreType.DMA,
            pltpu.SemaphoreType.DMA,
            pltpu.SemaphoreType.DMA,
            pltpu.SemaphoreType.DMA,
        ],
    )(_ball_body)


def _mlp1_body(g_ref, gx_ref, nx_ref, wp_ref, wx_ref, b_ref, z_ref, st_ref):
    g = g_ref[0]
    z = jnp.dot(g, wp_ref[...], preferred_element_type=jnp.float32)
    z = z + jnp.dot(gx_ref[0], wx_ref[...], preferred_element_type=jnp.float32)
    nx = nx_ref[0]
    corr = jnp.dot(nx, wx_ref[...], preferred_element_type=jnp.float32)
    kr = _TR // _NS
    corr = jnp.broadcast_to(corr[:, None, :], (kr, _NS, corr.shape[-1]))
    z = z - corr.reshape(_TR, -1) + b_ref[...]
    z_ref[0] = z

    @pl.when(jnp.logical_and(pl.program_id(0) == 0, pl.program_id(1) == 0))
    def _():
        st_ref[...] = jnp.zeros_like(st_ref)

    st_ref[0:1] = st_ref[0:1] + jnp.sum(z, axis=0, keepdims=True)
    st_ref[1:2] = st_ref[1:2] + jnp.sum(z * z, axis=0, keepdims=True)


def _bn(z, st_ref, gamma, beta):
    inv = jnp.float32(1.0 / _CNT)
    mean = st_ref[0:1] * inv
    var = st_ref[1:2] * inv - mean * mean
    rstd = lax.rsqrt(var + jnp.float32(1e-5))
    scale = rstd * gamma
    shift = beta - mean * scale
    return jnp.maximum(z * scale + shift, jnp.float32(0))


def _mlp_mid_body(z_ref, stp_ref, ga_ref, be_ref, w_ref, b_ref,
                  z2_ref, st_ref):
    a = _bn(z_ref[0], stp_ref, ga_ref[...], be_ref[...])
    z2 = jnp.dot(a, w_ref[...], preferred_element_type=jnp.float32)
    z2 = z2 + b_ref[...]
    z2_ref[0] = z2

    @pl.when(jnp.logical_and(pl.program_id(0) == 0, pl.program_id(1) == 0))
    def _():
        st_ref[...] = jnp.zeros_like(st_ref)

    st_ref[0:1] = st_ref[0:1] + jnp.sum(z2, axis=0, keepdims=True)
    st_ref[1:2] = st_ref[1:2] + jnp.sum(z2 * z2, axis=0, keepdims=True)


def _mlp_stat3_body(z_ref, stp_ref, ga_ref, be_ref, m_ref, sa_ref):
    a = _bn(z_ref[0], stp_ref, ga_ref[...], be_ref[...])

    @pl.when(jnp.logical_and(pl.program_id(0) == 0, pl.program_id(1) == 0))
    def _():
        m_ref[...] = jnp.zeros_like(m_ref)
        sa_ref[...] = jnp.zeros_like(sa_ref)

    m_ref[...] = m_ref[...] + lax.dot_general(
        a, a, (((0,), (0,)), ((), ())), preferred_element_type=jnp.float32)
    sa_ref[0:1] = sa_ref[0:1] + jnp.sum(a, axis=0, keepdims=True)


def _mlp_final_body(z_ref, stp_ref, ga1_ref, be1_ref, w_ref, b_ref,
                    m_ref, sa_ref, ga2_ref, be2_ref, out_ref):
    a = _bn(z_ref[0], stp_ref, ga1_ref[...], be1_ref[...])
    w = w_ref[...]
    z3 = jnp.dot(a, w, preferred_element_type=jnp.float32) + b_ref[...]
    inv = jnp.float32(1.0 / _CNT)
    mnb = jnp.dot(sa_ref[0:1], w, preferred_element_type=jnp.float32) * inv
    t1 = jnp.dot(m_ref[...], w, preferred_element_type=jnp.float32)
    q = jnp.sum(w * t1, axis=0, keepdims=True) * inv
    mean3 = mnb + b_ref[...]
    var3 = q - mnb * mnb
    rstd = lax.rsqrt(var3 + jnp.float32(1e-5))
    scale = rstd * ga2_ref[...]
    shift = be2_ref[...] - mean3 * scale
    a3 = jnp.maximum(z3 * scale + shift, jnp.float32(0))
    c = a3.shape[-1]
    a3 = a3.reshape(_TR // _NS, _NS, c)
    out_ref[0] = jnp.max(a3, axis=1)


def _run_mlp(gpts, gxyz, new_xyz, w0pt, w0xt, b0, g0, be0, w1t, b1, g1, be1,
             w2t, b2, g2, be2):
    c1, c2, c3 = w0pt.shape[1], w1t.shape[1], w2t.shape[1]
    kr = _TR // _NS

    z1, st1 = pl.pallas_call(
        _mlp1_body,
        grid=(_B, _NT),
        in_specs=[
            pl.BlockSpec((1, _TR, _D), lambda bi, t: (bi, t, 0)),
            pl.BlockSpec((1, _TR, 3), lambda bi, t: (bi, t, 0)),
            pl.BlockSpec((1, kr, 3), lambda bi, t: (bi, t, 0)),
            pl.BlockSpec((_D, c1), lambda bi, t: (0, 0)),
            pl.BlockSpec((3, c1), lambda bi, t: (0, 0)),
            pl.BlockSpec((1, c1), lambda bi, t: (0, 0)),
        ],
        out_specs=[
            pl.BlockSpec((1, _TR, c1), lambda bi, t: (bi, t, 0)),
            pl.BlockSpec((8, c1), lambda bi, t: (0, 0)),
        ],
        out_shape=[
            jax.ShapeDtypeStruct((_B, _NP * _NS, c1), jnp.float32),
            jax.ShapeDtypeStruct((8, c1), jnp.float32),
        ],
    )(gpts, gxyz, new_xyz, w0pt, w0xt, b0[None])

    z2, st2 = pl.pallas_call(
        _mlp_mid_body,
        grid=(_B, _NT),
        in_specs=[
            pl.BlockSpec((1, _TR, c1), lambda bi, t: (bi, t, 0)),
            pl.BlockSpec((8, c1), lambda bi, t: (0, 0)),
            pl.BlockSpec((1, c1), lambda bi, t: (0, 0)),
            pl.BlockSpec((1, c1), lambda bi, t: (0, 0)),
            pl.BlockSpec((c1, c2), lambda bi, t: (0, 0)),
            pl.BlockSpec((1, c2), lambda bi, t: (0, 0)),
        ],
        out_specs=[
            pl.BlockSpec((1, _TR, c2), lambda bi, t: (bi, t, 0)),
            pl.BlockSpec((8, c2), lambda bi, t: (0, 0)),
        ],
        out_shape=[
            jax.ShapeDtypeStruct((_B, _NP * _NS, c2), jnp.float32),
            jax.ShapeDtypeStruct((8, c2), jnp.float32),
        ],
    )(z1, st1, g0[None], be0[None], w1t, b1[None])

    m3, sa3 = pl.pallas_call(
        _mlp_stat3_body,
        grid=(_B, _NT),
        in_specs=[
            pl.BlockSpec((1, _TR, c2), lambda bi, t: (bi, t, 0)),
            pl.BlockSpec((8, c2), lambda bi, t: (0, 0)),
            pl.BlockSpec((1, c2), lambda bi, t: (0, 0)),
            pl.BlockSpec((1, c2), lambda bi, t: (0, 0)),
        ],
        out_specs=[
            pl.BlockSpec((c2, c2), lambda bi, t: (0, 0)),
            pl.BlockSpec((8, c2), lambda bi, t: (0, 0)),
        ],
        out_shape=[
            jax.ShapeDtypeStruct((c2, c2), jnp.float32),
            jax.ShapeDtypeStruct((8, c2), jnp.float32),
        ],
    )(z2, st2, g1[None], be1[None])

    new_points = pl.pallas_call(
        _mlp_final_body,
        grid=(_B, _NT),
        in_specs=[
            pl.BlockSpec((1, _TR, c2), lambda bi, t: (bi, t, 0)),
            pl.BlockSpec((8, c2), lambda bi, t: (0, 0)),
            pl.BlockSpec((1, c2), lambda bi, t: (0, 0)),
            pl.BlockSpec((1, c2), lambda bi, t: (0, 0)),
            pl.BlockSpec((c2, c3), lambda bi, t: (0, 0)),
            pl.BlockSpec((1, c3), lambda bi, t: (0, 0)),
            pl.BlockSpec((c2, c2), lambda bi, t: (0, 0)),
            pl.BlockSpec((8, c2), lambda bi, t: (0, 0)),
            pl.BlockSpec((1, c3), lambda bi, t: (0, 0)),
            pl.BlockSpec((1, c3), lambda bi, t: (0, 0)),
        ],
        out_specs=pl.BlockSpec((1, kr, c3), lambda bi, t: (bi, t, 0)),
        out_shape=jax.ShapeDtypeStruct((_B, _NP, c3), jnp.float32),
    )(z2, st2, g1[None], be1[None], w2t, b2[None], m3, sa3,
      g2[None], be2[None])
    return new_points


def kernel(xyz, points, W0, b0, gamma0, beta0, W1, b1, gamma1, beta1,
           W2, b2, gamma2, beta2):
    xyzp = jnp.transpose(xyz, (2, 0, 1))
    nxyz_t = _run_fps(xyzp)
    new_xyz = jnp.transpose(nxyz_t, (1, 0, 2))

    dist = _run_dist(new_xyz, jnp.transpose(xyz, (0, 2, 1)))
    dist = dist.reshape(_B * _NP, _N)

    pts_table = points.reshape(_B * _N, _D)
    gpts, gxyz = _ball_kernel()(dist, xyzp[0], xyzp[1], xyzp[2], pts_table)
    gpts = gpts.reshape(_B, _NP * _NS, _D)
    gxyz = gxyz.reshape(_B, _NP * _NS, 3)

    new_points = _run_mlp(gpts, gxyz, new_xyz, W0[:, 3:].T, W0[:, :3].T,
                          b0, gamma0, beta0,
                          W1.T, b1, gamma1, beta1, W2.T, b2, gamma2, beta2)
    return (new_xyz, new_points)
